# Initial kernel scaffold; baseline (speedup 1.0000x reference)
import jax
import jax.numpy as jnp
from jax.experimental import pallas as pl


def kernel(X, A, edge_index, edge_weight, W1, b1, W2, b2):
    raise NotImplementedError("write your pallas kernel here")


# speedup vs baseline: 24.0484x; 24.0484x over previous
import jax
import jax.numpy as jnp
from jax import lax
from jax.experimental import pallas as pl
from jax.experimental.pallas import tpu as pltpu
from jax.experimental.pallas import tpu_sc as plsc

N = 10000
NPAD = 10240
E = 160000
HCH = 96
NC, NS = 2, 16
RPT = N // NS
EPT = E // NS
EK = 80
NCHUNK = EPT // EK
RK = 125
DPT = NPAD // NS


def _bcast16(ref, *idx):
    return plsc.load_gather(ref, [jnp.full((16,), i, jnp.int32) for i in idx])


def _vec_rsqrt(v):
    x = jnp.maximum(v, 1e-12)
    i = lax.bitcast_convert_type(x, jnp.int32)
    y = lax.bitcast_convert_type(jnp.int32(0x5F3759DF) - (i >> 1), jnp.float32)
    for _ in range(3):
        y = y * (1.5 - 0.5 * x * y * y)
    return jnp.where(v > 0, y, 0.0)


def _mm_body(x_ref, w_ref, o_ref):
    o_ref[...] = jnp.dot(x_ref[...], w_ref[...],
                         preferred_element_type=jnp.float32)


def _mm2_body(p_ref, w_ref, b_ref, o_ref):
    y = jnp.dot(p_ref[...], w_ref[...], preferred_element_type=jnp.float32)
    y = y + b_ref[...]
    o_ref[...] = 1.0 / (1.0 + jnp.exp(-y))


def _gcn_sc_body(h1_p0, h1_p1, src3, dst3, ew3, b1t,
                 t2_p0, t2_p1, pf_p0, pf_p1,
                 acc_sh, dd_sh,
                 dinv_v, src2, dst2, ew2, rows_a, t2buf, obuf, b1_v,
                 sem_a, sem_s):
    c = lax.axis_index("c")
    s = lax.axis_index("s")
    coff = c * N
    pltpu.sync_copy(b1t, b1_v)

    ones16 = jnp.full((16,), 1.0, jnp.float32)
    def fill_ones(j, _):
        obuf[pl.ds(j * 16, 16)] = ones16
        return 0
    lax.fori_loop(0, DPT // 16, fill_ones, 0)
    pltpu.sync_copy(obuf, dd_sh.at[pl.ds(s * DPT, DPT)])

    pltpu.sync_copy(dst3.at[s], dst2)
    pltpu.sync_copy(ew3.at[s], ew2)
    plsc.subcore_barrier()

    def deg_chunk(i, _):
        base = i * 5
        descs = [pltpu.async_copy(ew2.at[base + k], dd_sh.at[dst2.at[base + k]],
                                  sem_s, add=True)
                 for k in range(5)]
        for d in descs:
            d.wait()
        return 0
    lax.fori_loop(0, NCHUNK // 5, deg_chunk, 0)
    plsc.subcore_barrier()

    pltpu.sync_copy(dd_sh.at[pl.ds(s * DPT, DPT)], obuf)
    def dinv_chunk(j, _):
        sl = pl.ds(j * 16, 16)
        obuf[sl] = _vec_rsqrt(obuf[sl])
        return 0
    lax.fori_loop(0, DPT // 16, dinv_chunk, 0)
    pltpu.sync_copy(obuf, dd_sh.at[pl.ds(s * DPT, DPT)])
    plsc.subcore_barrier()
    pltpu.sync_copy(dd_sh, dinv_v)

    pltpu.sync_copy(src3.at[s], src2)
    def wpass(i, _):
        for k in range(5):
            sl = pl.ds(k * 16, 16)
            sv = src2[i, sl]
            ew2[i, sl] = ew2[i, sl] * plsc.load_gather(dinv_v, [sv])
            src2[i, sl] = sv + coff
        return 0
    lax.fori_loop(0, NCHUNK, wpass, 0)

    def row_scale(rb):
        def rowfn(r, _2):
            d = _bcast16(dinv_v, rb + r)
            for t in range(6):
                sl = pl.ds(t * 16, 16)
                t2buf[r, sl] = t2buf[r, sl] * d
            return 0
        lax.fori_loop(0, RK, rowfn, 0)

    def edge_pass(table_hbm):
        def body(i, _):
            pltpu.async_copy(table_hbm.at[src2.at[i]], rows_a, sem_a).wait()
            def rowfn(j, _2):
                w = _bcast16(ew2, i, j)
                for t in range(6):
                    sl = pl.ds(t * 16, 16)
                    rows_a[j, sl] = rows_a[j, sl] * w
                return 0
            lax.fori_loop(0, EK, rowfn, 0)
            pltpu.sync_copy(rows_a, acc_sh.at[dst2.at[i]], add=True)
            return 0
        lax.fori_loop(0, NCHUNK, body, 0)

    for p, hp, t2p, pfp in ((0, h1_p0, t2_p0, pf_p0),
                            (1, h1_p1, t2_p1, pf_p1)):
        def init_chunk(k, _):
            rb = s * RPT + k * RK
            pltpu.sync_copy(hp.at[pl.ds(coff + rb, RK)], t2buf)
            row_scale(rb)
            pltpu.sync_copy(t2buf, acc_sh.at[pl.ds(rb, RK)])
            return 0
        lax.fori_loop(0, RPT // RK, init_chunk, 0)
        plsc.subcore_barrier()
        edge_pass(hp)
        plsc.subcore_barrier()

        def wb1(k, _):
            rb = s * RPT + k * RK
            pltpu.sync_copy(acc_sh.at[pl.ds(rb, RK)], t2buf)
            def rowfn(r, _2):
                d = _bcast16(dinv_v, rb + r)
                for t in range(6):
                    sl = pl.ds(t * 16, 16)
                    t2buf[r, sl] = jnp.maximum(
                        t2buf[r, sl] * d + b1_v[pl.ds(p * HCH + t * 16, 16)],
                        0.0)
                return 0
            lax.fori_loop(0, RK, rowfn, 0)
            pltpu.sync_copy(t2buf, t2p.at[pl.ds(coff + rb, RK)])
            row_scale(rb)
            pltpu.sync_copy(t2buf, acc_sh.at[pl.ds(rb, RK)])
            return 0
        lax.fori_loop(0, RPT // RK, wb1, 0)
        plsc.subcore_barrier()

        edge_pass(t2p)
        plsc.subcore_barrier()

        def wb2(k, _):
            rb = s * RPT + k * RK
            pltpu.sync_copy(acc_sh.at[pl.ds(rb, RK)], t2buf)
            row_scale(rb)
            pltpu.sync_copy(t2buf, pfp.at[pl.ds(coff + rb, RK)])
            return 0
        lax.fori_loop(0, RPT // RK, wb2, 0)


@jax.jit
def kernel(X, A, edge_index, edge_weight, W1, b1, W2, b2):
    del A
    Bx, n, Tx, Cin = X.shape

    Xf = X.reshape(Bx * n * Tx, Cin)
    MB = 15000
    h1f = pl.pallas_call(
        _mm_body,
        grid=(Xf.shape[0] // MB,),
        in_specs=[pl.BlockSpec((MB, Cin), lambda i: (i, 0)),
                  pl.BlockSpec((Cin, 16), lambda i: (0, 0))],
        out_specs=pl.BlockSpec((MB, 16), lambda i: (i, 0)),
        out_shape=jax.ShapeDtypeStruct((Xf.shape[0], 16), jnp.float32),
    )(Xf, W1)
    h1r = h1f.reshape(Bx * n, 2, HCH)
    h1_p0, h1_p1 = h1r[:, 0], h1r[:, 1]

    src3 = edge_index[0].reshape(NS, NCHUNK, EK)
    dst3 = edge_index[1].reshape(NS, NCHUNK, EK)
    ew3 = edge_weight.reshape(NS, NCHUNK, EK)
    b1t = jnp.tile(b1, Tx)

    mesh = plsc.VectorSubcoreMesh(core_axis_name="c", subcore_axis_name="s",
                                  num_cores=NC, num_subcores=NS)
    half = jax.ShapeDtypeStruct((Bx * n, HCH), jnp.float32)
    t2_p0, t2_p1, pf_p0, pf_p1 = pl.kernel(
        _gcn_sc_body,
        out_type=(half, half, half, half),
        mesh=mesh,
        compiler_params=pltpu.CompilerParams(use_tc_tiling_on_sc=False,
                                             needs_layout_passes=False),
        scratch_types=(
            pltpu.VMEM_SHARED((N, HCH), jnp.float32),
            pltpu.VMEM_SHARED((NPAD,), jnp.float32),
            pltpu.VMEM((NPAD,), jnp.float32),
            pltpu.VMEM((NCHUNK, EK), jnp.int32),
            pltpu.VMEM((NCHUNK, EK), jnp.int32),
            pltpu.VMEM((NCHUNK, EK), jnp.float32),
            pltpu.VMEM((EK, HCH), jnp.float32),
            pltpu.VMEM((RK, HCH), jnp.float32),
            pltpu.VMEM((DPT,), jnp.float32),
            pltpu.VMEM((2 * HCH,), jnp.float32),
            pltpu.SemaphoreType.DMA,
            pltpu.SemaphoreType.DMA,
        ),
    )(h1_p0, h1_p1, src3, dst3, ew3, b1t)

    outs = []
    for pfp in (pf_p0, pf_p1):
        pr = pfp.reshape(Bx * n * (Tx // 2), 16)
        o = pl.pallas_call(
            _mm2_body,
            grid=(pr.shape[0] // MB,),
            in_specs=[pl.BlockSpec((MB, 16), lambda i: (i, 0)),
                      pl.BlockSpec((16, 16), lambda i: (0, 0)),
                      pl.BlockSpec((1, 16), lambda i: (0, 0))],
            out_specs=pl.BlockSpec((MB, 16), lambda i: (i, 0)),
            out_shape=jax.ShapeDtypeStruct((pr.shape[0], 16), jnp.float32),
        )(pr, W2, b2.reshape(1, 16))
        outs.append(o.reshape(Bx, n, Tx // 2, 16))
    return jnp.concatenate(outs, axis=2)


# speedup vs baseline: 32.5139x; 1.3520x over previous
import jax
import jax.numpy as jnp
from jax import lax
from jax.experimental import pallas as pl
from jax.experimental.pallas import tpu as pltpu
from jax.experimental.pallas import tpu_sc as plsc

N = 10000
NPAD = 10240
E = 160000
HCH = 96
NC, NS = 2, 16
RPT = N // NS
EPT = E // NS
EK = 80
NCHUNK = EPT // EK
RK = 125
DPT = NPAD // NS


def _bcast16(ref, *idx):
    return plsc.load_gather(ref, [jnp.full((16,), i, jnp.int32) for i in idx])


def _vec_rsqrt(v):
    x = jnp.maximum(v, 1e-12)
    i = lax.bitcast_convert_type(x, jnp.int32)
    y = lax.bitcast_convert_type(jnp.int32(0x5F3759DF) - (i >> 1), jnp.float32)
    for _ in range(3):
        y = y * (1.5 - 0.5 * x * y * y)
    return jnp.where(v > 0, y, 0.0)


def _mm_body(x_ref, w_ref, o_ref):
    o_ref[...] = jnp.dot(x_ref[...], w_ref[...],
                         preferred_element_type=jnp.float32)


def _mm2_body(p_ref, w_ref, b_ref, o_ref):
    y = jnp.dot(p_ref[...], w_ref[...], preferred_element_type=jnp.float32)
    y = y + b_ref[...]
    o_ref[...] = 1.0 / (1.0 + jnp.exp(-y))


def _gcn_sc_body(h1_p0, h1_p1, src3, dst3, ew3, b1t,
                 t2_p0, t2_p1, pf_p0, pf_p1,
                 acc_sh, dd_sh,
                 dinv_v, src2, dst2, ew2, rows_a, rows_b, t2buf, obuf, b1_v,
                 sem_a, sem_b, sem_s):
    c = lax.axis_index("c")
    s = lax.axis_index("s")
    coff = c * N
    pltpu.sync_copy(b1t, b1_v)

    ones16 = jnp.full((16,), 1.0, jnp.float32)
    def fill_ones(j, _):
        obuf[pl.ds(j * 16, 16)] = ones16
        return 0
    lax.fori_loop(0, DPT // 16, fill_ones, 0)
    pltpu.sync_copy(obuf, dd_sh.at[pl.ds(s * DPT, DPT)])

    pltpu.sync_copy(dst3.at[s], dst2)
    pltpu.sync_copy(ew3.at[s], ew2)
    plsc.subcore_barrier()

    def deg_chunk(i, _):
        base = i * 5
        descs = [pltpu.async_copy(ew2.at[base + k], dd_sh.at[dst2.at[base + k]],
                                  sem_s, add=True)
                 for k in range(5)]
        for d in descs:
            d.wait()
        return 0
    lax.fori_loop(0, NCHUNK // 5, deg_chunk, 0)
    plsc.subcore_barrier()

    pltpu.sync_copy(dd_sh.at[pl.ds(s * DPT, DPT)], obuf)
    def dinv_chunk(j, _):
        sl = pl.ds(j * 16, 16)
        obuf[sl] = _vec_rsqrt(obuf[sl])
        return 0
    lax.fori_loop(0, DPT // 16, dinv_chunk, 0)
    pltpu.sync_copy(obuf, dd_sh.at[pl.ds(s * DPT, DPT)])
    plsc.subcore_barrier()
    pltpu.sync_copy(dd_sh, dinv_v)

    pltpu.sync_copy(src3.at[s], src2)
    def wpass(i, _):
        for k in range(5):
            sl = pl.ds(k * 16, 16)
            sv = src2[i, sl]
            ew2[i, sl] = ew2[i, sl] * plsc.load_gather(dinv_v, [sv])
            src2[i, sl] = sv + coff
        return 0
    lax.fori_loop(0, NCHUNK, wpass, 0)

    def row_scale(rb):
        def rowfn(r5, _2):
            for u in range(5):
                r = r5 * 5 + u
                d = _bcast16(dinv_v, rb + r)
                for t in range(6):
                    sl = pl.ds(t * 16, 16)
                    t2buf[r, sl] = t2buf[r, sl] * d
            return 0
        lax.fori_loop(0, RK // 5, rowfn, 0)

    def edge_pass(table_hbm):
        def scale_scatter(buf, i):
            def rowfn(j4, _2):
                for u in range(4):
                    j = j4 * 4 + u
                    w = _bcast16(ew2, i, j)
                    for t in range(6):
                        sl = pl.ds(t * 16, 16)
                        buf[j, sl] = buf[j, sl] * w
                return 0
            lax.fori_loop(0, EK // 4, rowfn, 0)
            pltpu.sync_copy(buf, acc_sh.at[dst2.at[i]], add=True)

        pltpu.async_copy(table_hbm.at[src2.at[0]], rows_a, sem_a)
        def body(k, _):
            ia = 2 * k
            pltpu.async_copy(table_hbm.at[src2.at[ia + 1]], rows_b, sem_b)
            pltpu.make_async_copy(table_hbm.at[src2.at[ia]], rows_a,
                                  sem_a).wait()
            scale_scatter(rows_a, ia)
            pltpu.async_copy(table_hbm.at[src2.at[ia + 2]], rows_a, sem_a)
            pltpu.make_async_copy(table_hbm.at[src2.at[ia + 1]], rows_b,
                                  sem_b).wait()
            scale_scatter(rows_b, ia + 1)
            return 0
        lax.fori_loop(0, (NCHUNK - 1) // 2, body, 0)
        pltpu.make_async_copy(table_hbm.at[src2.at[NCHUNK - 1]], rows_a,
                              sem_a).wait()
        scale_scatter(rows_a, NCHUNK - 1)

    for p, hp, t2p, pfp in ((0, h1_p0, t2_p0, pf_p0),
                            (1, h1_p1, t2_p1, pf_p1)):
        def init_chunk(k, _):
            rb = s * RPT + k * RK
            pltpu.sync_copy(hp.at[pl.ds(coff + rb, RK)], t2buf)
            row_scale(rb)
            pltpu.sync_copy(t2buf, acc_sh.at[pl.ds(rb, RK)])
            return 0
        lax.fori_loop(0, RPT // RK, init_chunk, 0)
        plsc.subcore_barrier()
        edge_pass(hp)
        plsc.subcore_barrier()

        def wb1(k, _):
            rb = s * RPT + k * RK
            pltpu.sync_copy(acc_sh.at[pl.ds(rb, RK)], t2buf)
            def rowfn(r5, _2):
                for u in range(5):
                    r = r5 * 5 + u
                    d = _bcast16(dinv_v, rb + r)
                    for t in range(6):
                        sl = pl.ds(t * 16, 16)
                        t2buf[r, sl] = jnp.maximum(
                            t2buf[r, sl] * d
                            + b1_v[pl.ds(p * HCH + t * 16, 16)], 0.0)
                return 0
            lax.fori_loop(0, RK // 5, rowfn, 0)
            pltpu.sync_copy(t2buf, t2p.at[pl.ds(coff + rb, RK)])
            row_scale(rb)
            pltpu.sync_copy(t2buf, acc_sh.at[pl.ds(rb, RK)])
            return 0
        lax.fori_loop(0, RPT // RK, wb1, 0)
        plsc.subcore_barrier()

        edge_pass(t2p)
        plsc.subcore_barrier()

        def wb2(k, _):
            rb = s * RPT + k * RK
            pltpu.sync_copy(acc_sh.at[pl.ds(rb, RK)], t2buf)
            row_scale(rb)
            pltpu.sync_copy(t2buf, pfp.at[pl.ds(coff + rb, RK)])
            return 0
        lax.fori_loop(0, RPT // RK, wb2, 0)


@jax.jit
def kernel(X, A, edge_index, edge_weight, W1, b1, W2, b2):
    del A
    Bx, n, Tx, Cin = X.shape

    Xf = X.reshape(Bx * n * Tx, Cin)
    MB = 15000
    h1f = pl.pallas_call(
        _mm_body,
        grid=(Xf.shape[0] // MB,),
        in_specs=[pl.BlockSpec((MB, Cin), lambda i: (i, 0)),
                  pl.BlockSpec((Cin, 16), lambda i: (0, 0))],
        out_specs=pl.BlockSpec((MB, 16), lambda i: (i, 0)),
        out_shape=jax.ShapeDtypeStruct((Xf.shape[0], 16), jnp.float32),
    )(Xf, W1)
    h1r = h1f.reshape(Bx * n, 2, HCH)
    h1_p0, h1_p1 = h1r[:, 0], h1r[:, 1]

    src3 = edge_index[0].reshape(NS, NCHUNK, EK)
    dst3 = edge_index[1].reshape(NS, NCHUNK, EK)
    ew3 = edge_weight.reshape(NS, NCHUNK, EK)
    b1t = jnp.tile(b1, Tx)

    mesh = plsc.VectorSubcoreMesh(core_axis_name="c", subcore_axis_name="s",
                                  num_cores=NC, num_subcores=NS)
    half = jax.ShapeDtypeStruct((Bx * n, HCH), jnp.float32)
    t2_p0, t2_p1, pf_p0, pf_p1 = pl.kernel(
        _gcn_sc_body,
        out_type=(half, half, half, half),
        mesh=mesh,
        compiler_params=pltpu.CompilerParams(use_tc_tiling_on_sc=False,
                                             needs_layout_passes=False),
        scratch_types=(
            pltpu.VMEM_SHARED((N, HCH), jnp.float32),
            pltpu.VMEM_SHARED((NPAD,), jnp.float32),
            pltpu.VMEM((NPAD,), jnp.float32),
            pltpu.VMEM((NCHUNK, EK), jnp.int32),
            pltpu.VMEM((NCHUNK, EK), jnp.int32),
            pltpu.VMEM((NCHUNK, EK), jnp.float32),
            pltpu.VMEM((EK, HCH), jnp.float32),
            pltpu.VMEM((EK, HCH), jnp.float32),
            pltpu.VMEM((RK, HCH), jnp.float32),
            pltpu.VMEM((DPT,), jnp.float32),
            pltpu.VMEM((2 * HCH,), jnp.float32),
            pltpu.SemaphoreType.DMA,
            pltpu.SemaphoreType.DMA,
            pltpu.SemaphoreType.DMA,
        ),
    )(h1_p0, h1_p1, src3, dst3, ew3, b1t)

    outs = []
    for pfp in (pf_p0, pf_p1):
        pr = pfp.reshape(Bx * n * (Tx // 2), 16)
        o = pl.pallas_call(
            _mm2_body,
            grid=(pr.shape[0] // MB,),
            in_specs=[pl.BlockSpec((MB, 16), lambda i: (i, 0)),
                      pl.BlockSpec((16, 16), lambda i: (0, 0)),
                      pl.BlockSpec((1, 16), lambda i: (0, 0))],
            out_specs=pl.BlockSpec((MB, 16), lambda i: (i, 0)),
            out_shape=jax.ShapeDtypeStruct((pr.shape[0], 16), jnp.float32),
        )(pr, W2, b2.reshape(1, 16))
        outs.append(o.reshape(Bx, n, Tx // 2, 16))
    return jnp.concatenate(outs, axis=2)
